# speedup vs baseline: 5.5071x; 5.5071x over previous
import jax
import jax.numpy as jnp
from jax import lax
from jax.experimental import pallas as pl
from jax.experimental.pallas import tpu as pltpu
from jax.experimental.pallas import tpu_sc as plsc

EMBED = 128
B_TOTAL = 1024 * 200
NC, NS, L = 2, 16, 16
NW = NC * NS
B_PER_W = B_TOTAL // NW
CHUNK = 128
NCHUNK = B_PER_W // CHUNK


def _gather_kernel(d1_hbm, d2_hbm, table_hbm, out_hbm,
                   d1_v, d2_v, idx_v, rows_v, sem):
    wid = lax.axis_index("s") * NC + lax.axis_index("c")
    base = wid * B_PER_W

    pltpu.sync_copy(d1_hbm.at[pl.ds(base, B_PER_W)], d1_v)
    pltpu.sync_copy(d2_hbm.at[pl.ds(base, B_PER_W)], d2_v)

    def compute_idx(c, carry):
        for i in range(CHUNK // L):
            s = c * CHUNK + i * L
            idx_v[c, pl.ds(i * L, L)] = (
                d1_v[pl.ds(s, L)] * 256 + d2_v[pl.ds(s, L)])
        return carry

    lax.fori_loop(0, NCHUNK, compute_idx, 0, unroll=False)

    def gather_chunk(c, carry):
        pltpu.async_copy(table_hbm.at[idx_v.at[c]], rows_v, sem).wait()
        pltpu.sync_copy(rows_v, out_hbm.at[pl.ds(base + c * CHUNK, CHUNK)])
        return carry

    lax.fori_loop(0, NCHUNK, gather_chunk, 0, unroll=False)


def kernel(dim1_indices, dim2_indices, pos_embed):
    d1 = dim1_indices.reshape(-1)
    d2 = dim2_indices.reshape(-1)

    k = pl.kernel(
        _gather_kernel,
        out_type=jax.ShapeDtypeStruct((B_TOTAL, EMBED), jnp.float32),
        mesh=plsc.VectorSubcoreMesh(core_axis_name="c", subcore_axis_name="s"),
        scratch_types=[
            pltpu.VMEM((B_PER_W,), jnp.int32),
            pltpu.VMEM((B_PER_W,), jnp.int32),
            pltpu.VMEM((NCHUNK, CHUNK), jnp.int32),
            pltpu.VMEM((CHUNK, EMBED), jnp.float32),
            pltpu.SemaphoreType.DMA,
        ],
    )
    out = k(d1, d2, pos_embed)
    return out.reshape(dim1_indices.shape[0], dim1_indices.shape[1], EMBED)


# speedup vs baseline: 7.6472x; 1.3886x over previous
import jax
import jax.numpy as jnp
from jax import lax
from jax.experimental import pallas as pl
from jax.experimental.pallas import tpu as pltpu
from jax.experimental.pallas import tpu_sc as plsc

EMBED = 128
B_TOTAL = 1024 * 200
NC, NS, L = 2, 16, 16
NW = NC * NS
B_PER_W = B_TOTAL // NW
CHUNK = 128
NCHUNK = B_PER_W // CHUNK
NBUF = 5
AHEAD = 4


def _gather_kernel(d1_hbm, d2_hbm, table_hbm, out_hbm,
                   d1_v, d2_v, idx_v, rows_v, gsem, osem):
    wid = lax.axis_index("s") * NC + lax.axis_index("c")
    base = wid * B_PER_W

    pltpu.sync_copy(d1_hbm.at[pl.ds(base, B_PER_W)], d1_v)
    pltpu.sync_copy(d2_hbm.at[pl.ds(base, B_PER_W)], d2_v)

    def compute_idx(c, carry):
        for i in range(CHUNK // L):
            s = c * CHUNK + i * L
            idx_v[c, pl.ds(i * L, L)] = (
                d1_v[pl.ds(s, L)] * 256 + d2_v[pl.ds(s, L)])
        return carry

    lax.fori_loop(0, NCHUNK, compute_idx, 0, unroll=False)

    def fire_gather(ch, b):
        pltpu.async_copy(table_hbm.at[idx_v.at[ch]], rows_v.at[b], gsem)

    def wait_gather(b):
        pltpu.make_async_copy(
            table_hbm.at[pl.ds(0, CHUNK)], rows_v.at[b], gsem).wait()

    def fire_out(ch, b):
        pltpu.async_copy(
            rows_v.at[b], out_hbm.at[pl.ds(base + ch * CHUNK, CHUNK)], osem)

    def wait_out(b):
        pltpu.make_async_copy(
            rows_v.at[b], out_hbm.at[pl.ds(base, CHUNK)], osem).wait()

    for ch in range(AHEAD):
        fire_gather(ch, ch)

    for k in range(NBUF):
        if k >= 1:
            wait_out((k - 1) % NBUF)
        fire_gather(k + AHEAD, (k + AHEAD) % NBUF)
        wait_gather(k % NBUF)
        fire_out(k, k % NBUF)

    def step(c, carry):
        for k in range(NBUF):
            ch = c * NBUF + k
            wait_out((k - 1) % NBUF)
            fire_gather(ch + AHEAD, (k + AHEAD) % NBUF)
            wait_gather(k)
            fire_out(ch, k)
        return carry

    lax.fori_loop(1, NCHUNK // NBUF - 1, step, 0, unroll=False)

    for k in range(NBUF):
        ch = NCHUNK - NBUF + k
        wait_out((k - 1) % NBUF)
        if ch + AHEAD < NCHUNK:
            fire_gather(ch + AHEAD, (k + AHEAD) % NBUF)
        wait_gather(k)
        fire_out(ch, k)
    wait_out((NCHUNK - 1) % NBUF)


def kernel(dim1_indices, dim2_indices, pos_embed):
    d1 = dim1_indices.reshape(-1)
    d2 = dim2_indices.reshape(-1)

    k = pl.kernel(
        _gather_kernel,
        out_type=jax.ShapeDtypeStruct((B_TOTAL, EMBED), jnp.float32),
        mesh=plsc.VectorSubcoreMesh(core_axis_name="c", subcore_axis_name="s"),
        scratch_types=[
            pltpu.VMEM((B_PER_W,), jnp.int32),
            pltpu.VMEM((B_PER_W,), jnp.int32),
            pltpu.VMEM((NCHUNK, CHUNK), jnp.int32),
            pltpu.VMEM((NBUF, CHUNK, EMBED), jnp.float32),
            pltpu.SemaphoreType.DMA,
            pltpu.SemaphoreType.DMA,
        ],
    )
    out = k(d1, d2, pos_embed)
    return out.reshape(dim1_indices.shape[0], dim1_indices.shape[1], EMBED)


# speedup vs baseline: 7.6775x; 1.0040x over previous
import jax
import jax.numpy as jnp
from jax import lax
from jax.experimental import pallas as pl
from jax.experimental.pallas import tpu as pltpu
from jax.experimental.pallas import tpu_sc as plsc

EMBED = 128
B_TOTAL = 1024 * 200
NC, NS, L = 2, 16, 16
NW = NC * NS
B_PER_W = B_TOTAL // NW
CHUNK = 128
NCHUNK = B_PER_W // CHUNK
NBUF = 5
AHEAD = 4


def _gather_kernel(d1_hbm, d2_hbm, table_hbm, out_hbm,
                   d1_v, d2_v, idx_v, rows_v, gsem, osem):
    wid = lax.axis_index("s") * NC + lax.axis_index("c")
    base = wid * B_PER_W

    pltpu.sync_copy(d1_hbm.at[pl.ds(base, B_PER_W)], d1_v)
    pltpu.sync_copy(d2_hbm.at[pl.ds(base, B_PER_W)], d2_v)

    def compute_idx(c):
        for i in range(CHUNK // L):
            s = c * CHUNK + i * L
            idx_v[c, pl.ds(i * L, L)] = (
                d1_v[pl.ds(s, L)] * 256 + d2_v[pl.ds(s, L)])

    def fire_gather(ch, b):
        pltpu.async_copy(table_hbm.at[idx_v.at[ch]], rows_v.at[b], gsem)

    def wait_gather(b):
        pltpu.make_async_copy(
            table_hbm.at[pl.ds(0, CHUNK)], rows_v.at[b], gsem).wait()

    def fire_out(ch, b):
        pltpu.async_copy(
            rows_v.at[b], out_hbm.at[pl.ds(base + ch * CHUNK, CHUNK)], osem)

    def wait_out(b):
        pltpu.make_async_copy(
            rows_v.at[b], out_hbm.at[pl.ds(base, CHUNK)], osem).wait()

    for ch in range(AHEAD):
        compute_idx(ch)
        fire_gather(ch, ch)

    for k in range(NBUF):
        if k >= 1:
            wait_out((k - 1) % NBUF)
        compute_idx(k + AHEAD)
        fire_gather(k + AHEAD, (k + AHEAD) % NBUF)
        wait_gather(k % NBUF)
        fire_out(k, k % NBUF)

    def step(c, carry):
        for k in range(NBUF):
            ch = c * NBUF + k
            wait_out((k - 1) % NBUF)
            compute_idx(ch + AHEAD)
            fire_gather(ch + AHEAD, (k + AHEAD) % NBUF)
            wait_gather(k)
            fire_out(ch, k)
        return carry

    lax.fori_loop(1, NCHUNK // NBUF - 1, step, 0, unroll=False)

    for k in range(NBUF):
        ch = NCHUNK - NBUF + k
        wait_out((k - 1) % NBUF)
        if ch + AHEAD < NCHUNK:
            compute_idx(ch + AHEAD)
            fire_gather(ch + AHEAD, (k + AHEAD) % NBUF)
        wait_gather(k)
        fire_out(ch, k)
    wait_out((NCHUNK - 1) % NBUF)


def kernel(dim1_indices, dim2_indices, pos_embed):
    d1 = dim1_indices.reshape(-1)
    d2 = dim2_indices.reshape(-1)

    k = pl.kernel(
        _gather_kernel,
        out_type=jax.ShapeDtypeStruct((B_TOTAL, EMBED), jnp.float32),
        mesh=plsc.VectorSubcoreMesh(core_axis_name="c", subcore_axis_name="s"),
        scratch_types=[
            pltpu.VMEM((B_PER_W,), jnp.int32),
            pltpu.VMEM((B_PER_W,), jnp.int32),
            pltpu.VMEM((NCHUNK, CHUNK), jnp.int32),
            pltpu.VMEM((NBUF, CHUNK, EMBED), jnp.float32),
            pltpu.SemaphoreType.DMA,
            pltpu.SemaphoreType.DMA,
        ],
    )
    out = k(d1, d2, pos_embed)
    return out.reshape(dim1_indices.shape[0], dim1_indices.shape[1], EMBED)
